# baseline (speedup 1.0000x reference)
import functools

import jax
import jax.numpy as jnp
from jax.experimental import pallas as pl
from jax.experimental.pallas import tpu as pltpu

B = 128
V = 100000
VBLK = 4096
NBLK = (V + VBLK - 1) // VBLK


def _fused_body(x_ref, sample_ref, ent_ref, m_ref, s_ref, t_ref, bv_ref, bi_ref):
    j = pl.program_id(0)

    @pl.when(j == 0)
    def _init():
        m_ref[...] = jnp.full((B, 1), -jnp.inf, jnp.float32)
        s_ref[...] = jnp.zeros((B, 1), jnp.float32)
        t_ref[...] = jnp.zeros((B, 1), jnp.float32)
        bv_ref[...] = jnp.full((B, 1), -jnp.inf, jnp.float32)
        bi_ref[...] = jnp.zeros((B, 1), jnp.int32)

    x = x_ref[...]
    col = jax.lax.broadcasted_iota(jnp.int32, (B, VBLK), 1) + j * VBLK
    valid = col < V
    xm = jnp.where(valid, x, -jnp.inf)

    blk_max = jnp.max(xm, axis=1, keepdims=True)
    blk_arg = jnp.argmax(xm, axis=1).astype(jnp.int32).reshape(B, 1) + j * VBLK
    better = blk_max > bv_ref[...]
    bi_ref[...] = jnp.where(better, blk_arg, bi_ref[...])
    bv_ref[...] = jnp.where(better, blk_max, bv_ref[...])

    m_old = m_ref[...]
    m_new = jnp.maximum(m_old, blk_max)
    scale = jnp.exp(m_old - m_new)
    e = jnp.where(valid, jnp.exp(x - m_new), 0.0)
    x0 = jnp.where(valid, x, 0.0)
    s_new = s_ref[...] * scale + jnp.sum(e, axis=1, keepdims=True)
    t_new = t_ref[...] * scale + jnp.sum(x0 * e, axis=1, keepdims=True)
    m_ref[...] = m_new
    s_ref[...] = s_new
    t_ref[...] = t_new

    @pl.when(j == NBLK - 1)
    def _finish():
        ent_ref[...] = m_new + jnp.log(s_new) - t_new / s_new
        sample_ref[...] = bi_ref[...]


@functools.partial(jax.jit)
def _fused(logits):
    sample, ent = pl.pallas_call(
        _fused_body,
        grid=(NBLK,),
        in_specs=[pl.BlockSpec((B, VBLK), lambda j: (0, j))],
        out_specs=[
            pl.BlockSpec((B, 1), lambda j: (0, 0)),
            pl.BlockSpec((B, 1), lambda j: (0, 0)),
        ],
        out_shape=[
            jax.ShapeDtypeStruct((B, 1), jnp.int32),
            jax.ShapeDtypeStruct((B, 1), jnp.float32),
        ],
        scratch_shapes=[
            pltpu.VMEM((B, 1), jnp.float32),
            pltpu.VMEM((B, 1), jnp.float32),
            pltpu.VMEM((B, 1), jnp.float32),
            pltpu.VMEM((B, 1), jnp.float32),
            pltpu.VMEM((B, 1), jnp.int32),
        ],
    )(logits)
    return sample.reshape(B), ent.reshape(B)


def kernel(logits):
    sample, entropy = _fused(logits)
    return (sample, logits, entropy)
